# baseline (speedup 1.0000x reference)
import functools

import jax
import jax.numpy as jnp
from jax import lax
from jax.experimental import pallas as pl
from jax.experimental.pallas import tpu as pltpu
from jax.experimental.pallas import tpu_sc as plsc

L = 16
NUM_WORKERS = 32
CHUNK = 128
U_SLOTS = 4
V_SLOTS = 2


def _gmf_body(users_hbm, items_hbm, ut_hbm, it_hbm, out_hbm,
              idx, rows, sem_u, sem_v):
  n_chunks = idx.shape[0] // (2 * CHUNK)
  d = ut_hbm.shape[1]
  wid = lax.axis_index("s") * 2 + lax.axis_index("c")
  base = wid * n_chunks * CHUNK

  rest = (n_chunks - 1) * CHUNK
  half = n_chunks * CHUNK
  ci0 = pltpu.async_copy(users_hbm.at[pl.ds(base, CHUNK)],
                         idx.at[pl.ds(0, CHUNK)], sem_u)
  cj0 = pltpu.async_copy(items_hbm.at[pl.ds(base, CHUNK)],
                         idx.at[pl.ds(half, CHUNK)], sem_v)
  ci1 = pltpu.async_copy(users_hbm.at[pl.ds(base + CHUNK, rest)],
                         idx.at[pl.ds(CHUNK, rest)], sem_u)
  cj1 = pltpu.async_copy(items_hbm.at[pl.ds(base + CHUNK, rest)],
                         idx.at[pl.ds(half + CHUNK, rest)], sem_v)
  ci0.wait()
  cj0.wait()

  def fire_u(j):
    return pltpu.async_copy(ut_hbm.at[idx.at[pl.ds(j * CHUNK, CHUNK)]],
                            rows.at[j % U_SLOTS], sem_u)

  def fire_v(j, off=0, sz=CHUNK):
    return pltpu.async_copy(
        it_hbm.at[idx.at[pl.ds(half + j * CHUNK + off, sz)]],
        rows.at[U_SLOTS + j % V_SLOTS, pl.ds(off, sz)], sem_v)

  gu = [None] * n_chunks
  gv = [None] * n_chunks
  outs = [None] * n_chunks
  gu[0] = fire_u(0)
  gv[0] = fire_v(0)
  ci1.wait()
  cj1.wait()
  for j in range(1, min(max(U_SLOTS, V_SLOTS), n_chunks)):
    if j < U_SLOTS:
      gu[j] = fire_u(j)
    if j < V_SLOTS:
      gv[j] = fire_v(j)

  last = n_chunks - 1
  for j in range(n_chunks):
    gu[j].wait()
    uslot = j % U_SLOTS
    vslot = U_SLOTS + j % V_SLOTS

    def mul_rows(lo, hi):
      def mul_row(r, _):
        for k2 in range(d // L):
          s = pl.ds(k2 * L, L)
          rows[uslot, r, s] = rows[uslot, r, s] * rows[vslot, r, s]
        return _
      lax.fori_loop(lo, hi, mul_row, 0, unroll=4)

    if j == last:
      gv[j].wait()
      mul_rows(0, CHUNK - 32)
      gv_tail.wait()
      mul_rows(CHUNK - 32, CHUNK)
    else:
      gv[j].wait()
      mul_rows(0, CHUNK)
    k = j + V_SLOTS
    if k < n_chunks - 1:
      gv[k] = fire_v(k)
    elif k == n_chunks - 1:
      gv[k] = fire_v(k, 0, CHUNK - 32)
      gv_tail = fire_v(k, CHUNK - 32, 32)
    outs[j] = pltpu.async_copy(
        rows.at[uslot],
        out_hbm.at[pl.ds((wid * n_chunks + j) * CHUNK, CHUNK)], sem_u)

  for c in outs:
    if c is not None:
      c.wait()


def kernel(users, items, user_table, item_table):
  b = users.shape[0]
  d = user_table.shape[1]
  n_chunks = b // (NUM_WORKERS * CHUNK)

  mesh = plsc.VectorSubcoreMesh(core_axis_name="c", subcore_axis_name="s")
  run = functools.partial(
      pl.kernel,
      mesh=mesh,
      out_type=jax.ShapeDtypeStruct((b, d), jnp.float32),
      scratch_types=[
          pltpu.VMEM((2 * n_chunks * CHUNK,), jnp.int32),
          pltpu.VMEM((U_SLOTS + V_SLOTS, CHUNK, d), jnp.float32),
          pltpu.SemaphoreType.DMA,
          pltpu.SemaphoreType.DMA,
      ],
  )(_gmf_body)
  return run(users.astype(jnp.int32), items.astype(jnp.int32),
             user_table, item_table)


# speedup vs baseline: 1.0136x; 1.0136x over previous
import functools

import jax
import jax.numpy as jnp
from jax import lax
from jax.experimental import pallas as pl
from jax.experimental.pallas import tpu as pltpu
from jax.experimental.pallas import tpu_sc as plsc

L = 16
NUM_WORKERS = 32
CHUNK = 128
U_SLOTS = 4
V_SLOTS = 2


def _gmf_body(users_hbm, items_hbm, ut_hbm, it_hbm, out_hbm,
              idx, rows, sem_u, sem_v):
  n_chunks = idx.shape[0] // (2 * CHUNK)
  d = ut_hbm.shape[1]
  wid = lax.axis_index("s") * 2 + lax.axis_index("c")
  base = wid * n_chunks * CHUNK

  half = n_chunks * CHUNK
  ci = pltpu.async_copy(users_hbm.at[pl.ds(base, half)],
                        idx.at[pl.ds(0, half)], sem_u)
  cj = pltpu.async_copy(items_hbm.at[pl.ds(base, half)],
                        idx.at[pl.ds(half, half)], sem_v)
  ci.wait()
  cj.wait()

  def fire_u(j):
    return pltpu.async_copy(ut_hbm.at[idx.at[pl.ds(j * CHUNK, CHUNK)]],
                            rows.at[j % U_SLOTS], sem_u)

  def fire_v(j):
    return pltpu.async_copy(it_hbm.at[idx.at[pl.ds(half + j * CHUNK, CHUNK)]],
                            rows.at[U_SLOTS + j % V_SLOTS], sem_v)

  gu = [None] * n_chunks
  gv = [None] * n_chunks
  outs = [None] * n_chunks
  gu[0] = fire_u(0)
  gv[0] = fire_v(0)
  for j in range(1, min(max(U_SLOTS, V_SLOTS), n_chunks)):
    if j < U_SLOTS:
      gu[j] = fire_u(j)
    if j < V_SLOTS:
      gv[j] = fire_v(j)

  for j in range(n_chunks):
    gu[j].wait()
    gv[j].wait()
    uslot = j % U_SLOTS
    vslot = U_SLOTS + j % V_SLOTS

    def mul_row(r, _):
      for k2 in range(d // L):
        s = pl.ds(k2 * L, L)
        rows[uslot, r, s] = rows[uslot, r, s] * rows[vslot, r, s]
      return _

    lax.fori_loop(0, CHUNK, mul_row, 0, unroll=4)
    if j + V_SLOTS < n_chunks:
      gv[j + V_SLOTS] = fire_v(j + V_SLOTS)
    outs[j] = pltpu.async_copy(
        rows.at[uslot],
        out_hbm.at[pl.ds((wid * n_chunks + j) * CHUNK, CHUNK)], sem_u)

  for c in outs:
    if c is not None:
      c.wait()


def kernel(users, items, user_table, item_table):
  b = users.shape[0]
  d = user_table.shape[1]
  n_chunks = b // (NUM_WORKERS * CHUNK)

  mesh = plsc.VectorSubcoreMesh(core_axis_name="c", subcore_axis_name="s")
  run = functools.partial(
      pl.kernel,
      mesh=mesh,
      out_type=jax.ShapeDtypeStruct((b, d), jnp.float32),
      scratch_types=[
          pltpu.VMEM((2 * n_chunks * CHUNK,), jnp.int32),
          pltpu.VMEM((U_SLOTS + V_SLOTS, CHUNK, d), jnp.float32),
          pltpu.SemaphoreType.DMA,
          pltpu.SemaphoreType.DMA,
      ],
  )(_gmf_body)
  return run(users.astype(jnp.int32), items.astype(jnp.int32),
             user_table, item_table)


# speedup vs baseline: 1.0267x; 1.0128x over previous
import functools

import jax
import jax.numpy as jnp
from jax import lax
from jax.experimental import pallas as pl
from jax.experimental.pallas import tpu as pltpu
from jax.experimental.pallas import tpu_sc as plsc

L = 16
NUM_WORKERS = 32
CHUNK = 128
U_SLOTS = 4
V_SLOTS = 2


def _gmf_body(users_hbm, items_hbm, ut_hbm, it_hbm, out_hbm,
              idx, rows, sem_u, sem_v):
  n_chunks = idx.shape[0] // (2 * CHUNK)
  d = ut_hbm.shape[1]
  wid = lax.axis_index("s") * 2 + lax.axis_index("c")
  base = wid * n_chunks * CHUNK

  half = n_chunks * CHUNK
  ci = pltpu.async_copy(users_hbm.at[pl.ds(base, half)],
                        idx.at[pl.ds(0, half)], sem_u)
  cj = pltpu.async_copy(items_hbm.at[pl.ds(base, half)],
                        idx.at[pl.ds(half, half)], sem_v)
  ci.wait()
  cj.wait()

  def fire_u(j):
    return pltpu.async_copy(ut_hbm.at[idx.at[pl.ds(j * CHUNK, CHUNK)]],
                            rows.at[j % U_SLOTS], sem_u)

  def fire_v(j):
    return pltpu.async_copy(it_hbm.at[idx.at[pl.ds(half + j * CHUNK, CHUNK)]],
                            rows.at[U_SLOTS + j % V_SLOTS], sem_v)

  gu = [None] * n_chunks
  gv = [None] * n_chunks
  outs = [None] * n_chunks
  gu[0] = fire_u(0)
  gv[0] = fire_v(0)
  for j in range(1, min(max(U_SLOTS, V_SLOTS), n_chunks)):
    if j < U_SLOTS:
      gu[j] = fire_u(j)
    if j < V_SLOTS:
      gv[j] = fire_v(j)

  for j in range(n_chunks):
    gu[j].wait()
    gv[j].wait()
    uslot = j % U_SLOTS
    vslot = U_SLOTS + j % V_SLOTS

    def mul_row(r, _):
      for k2 in range(d // L):
        s = pl.ds(k2 * L, L)
        rows[uslot, r, s] = rows[uslot, r, s] * rows[vslot, r, s]
      return _

    lax.fori_loop(0, CHUNK, mul_row, 0, unroll=2)
    if j + V_SLOTS < n_chunks:
      gv[j + V_SLOTS] = fire_v(j + V_SLOTS)
    outs[j] = pltpu.async_copy(
        rows.at[uslot],
        out_hbm.at[pl.ds((wid * n_chunks + j) * CHUNK, CHUNK)], sem_u)

  for c in outs:
    if c is not None:
      c.wait()


def kernel(users, items, user_table, item_table):
  b = users.shape[0]
  d = user_table.shape[1]
  n_chunks = b // (NUM_WORKERS * CHUNK)

  mesh = plsc.VectorSubcoreMesh(core_axis_name="c", subcore_axis_name="s")
  run = functools.partial(
      pl.kernel,
      mesh=mesh,
      out_type=jax.ShapeDtypeStruct((b, d), jnp.float32),
      scratch_types=[
          pltpu.VMEM((2 * n_chunks * CHUNK,), jnp.int32),
          pltpu.VMEM((U_SLOTS + V_SLOTS, CHUNK, d), jnp.float32),
          pltpu.SemaphoreType.DMA,
          pltpu.SemaphoreType.DMA,
      ],
  )(_gmf_body)
  return run(users.astype(jnp.int32), items.astype(jnp.int32),
             user_table, item_table)


# speedup vs baseline: 1.0358x; 1.0089x over previous
import functools

import jax
import jax.numpy as jnp
from jax import lax
from jax.experimental import pallas as pl
from jax.experimental.pallas import tpu as pltpu
from jax.experimental.pallas import tpu_sc as plsc

L = 16
NUM_WORKERS = 32
CHUNK = 128
U_SLOTS = 4
V_SLOTS = 2


def _gmf_body(users_hbm, items_hbm, ut_hbm, it_hbm, out_hbm,
              idx, rows, sem_u, sem_v):
  n_chunks = idx.shape[0] // (2 * CHUNK)
  d = ut_hbm.shape[1]
  wid = lax.axis_index("s") * 2 + lax.axis_index("c")
  base = wid * n_chunks * CHUNK

  half = n_chunks * CHUNK
  ci = pltpu.async_copy(users_hbm.at[pl.ds(base, half)],
                        idx.at[pl.ds(0, half)], sem_u)
  cj = pltpu.async_copy(items_hbm.at[pl.ds(base, half)],
                        idx.at[pl.ds(half, half)], sem_v)
  ci.wait()
  cj.wait()

  def fire_u(j):
    return pltpu.async_copy(ut_hbm.at[idx.at[pl.ds(j * CHUNK, CHUNK)]],
                            rows.at[j % U_SLOTS], sem_u)

  def fire_v(j):
    return pltpu.async_copy(it_hbm.at[idx.at[pl.ds(half + j * CHUNK, CHUNK)]],
                            rows.at[U_SLOTS + j % V_SLOTS], sem_v)

  gu = [None] * n_chunks
  gv = [None] * n_chunks
  outs = [None] * n_chunks
  gu[0] = fire_u(0)
  gv[0] = fire_v(0)
  for j in range(1, min(max(U_SLOTS, V_SLOTS), n_chunks)):
    if j < U_SLOTS:
      gu[j] = fire_u(j)
    if j < V_SLOTS:
      gv[j] = fire_v(j)

  for j in range(n_chunks):
    gu[j].wait()
    gv[j].wait()
    uslot = j % U_SLOTS
    vslot = U_SLOTS + j % V_SLOTS

    def mul_row(r, _):
      for k2 in range(d // L):
        s = pl.ds(k2 * L, L)
        rows[uslot, r, s] = rows[uslot, r, s] * rows[vslot, r, s]
      return _

    lax.fori_loop(0, CHUNK, mul_row, 0)
    if j + V_SLOTS < n_chunks:
      gv[j + V_SLOTS] = fire_v(j + V_SLOTS)
    outs[j] = pltpu.async_copy(
        rows.at[uslot],
        out_hbm.at[pl.ds((wid * n_chunks + j) * CHUNK, CHUNK)], sem_u)

  for c in outs:
    if c is not None:
      c.wait()


def kernel(users, items, user_table, item_table):
  b = users.shape[0]
  d = user_table.shape[1]
  n_chunks = b // (NUM_WORKERS * CHUNK)

  mesh = plsc.VectorSubcoreMesh(core_axis_name="c", subcore_axis_name="s")
  run = functools.partial(
      pl.kernel,
      mesh=mesh,
      out_type=jax.ShapeDtypeStruct((b, d), jnp.float32),
      scratch_types=[
          pltpu.VMEM((2 * n_chunks * CHUNK,), jnp.int32),
          pltpu.VMEM((U_SLOTS + V_SLOTS, CHUNK, d), jnp.float32),
          pltpu.SemaphoreType.DMA,
          pltpu.SemaphoreType.DMA,
      ],
  )(_gmf_body)
  return run(users.astype(jnp.int32), items.astype(jnp.int32),
             user_table, item_table)
